# Initial kernel scaffold; baseline (speedup 1.0000x reference)
import jax
import jax.numpy as jnp
from jax.experimental import pallas as pl


def kernel(idx, table):
    raise NotImplementedError("write your pallas kernel here")


# speedup vs baseline: 1.0939x; 1.0939x over previous
import jax
import jax.numpy as jnp
from jax.experimental import pallas as pl
from jax.experimental.pallas import tpu as pltpu
from jax.experimental.pallas import tpu_sc as plsc

_B, _T, _V = 1024, 50, 1000
_VP = 1024
_N = _B * _T
_W = 40
_R = 512

_MESH = plsc.VectorSubcoreMesh(core_axis_name="c", subcore_axis_name="s")


def _sc_gather(tabp, idx3):
    @pl.kernel(
        out_type=jax.ShapeDtypeStruct((_N, _VP), tabp.dtype),
        mesh=_MESH,
    )
    def gather_kernel(table_hbm, idx_hbm, out_hbm):
        def body(idx_vmem, out_vmem):
            pltpu.sync_copy(table_hbm.at[idx_vmem.at[0, 0]], out_vmem)

        pltpu.emit_pipeline(
            body,
            grid=(_N // _W,),
            in_specs=[pl.BlockSpec((1, 1, _W), lambda i: (i, 0, 0))],
            out_specs=[pl.BlockSpec((_W, _VP), lambda i: (i, 0))],
            core_axis_name=("c", "s"),
            dimension_semantics=(pltpu.PARALLEL,),
        )(idx_hbm, out_hbm)

    return gather_kernel(tabp, idx3)


def _tc_depad(padded):
    return pl.pallas_call(
        lambda x_ref, o_ref: o_ref.__setitem__(
            (slice(None), slice(None)), x_ref[:, :_V]
        ),
        out_shape=jax.ShapeDtypeStruct((_N, _V), padded.dtype),
        grid=(_N // _R,),
        in_specs=[pl.BlockSpec((_R, _VP), lambda i: (i, 0))],
        out_specs=pl.BlockSpec((_R, _V), lambda i: (i, 0)),
    )(padded)


def kernel(idx, table):
    tabp = jnp.pad(table, ((0, 0), (0, _VP - _V)))
    idx3 = idx.reshape(_N // _W, 1, _W)
    padded = _sc_gather(tabp, idx3)
    return _tc_depad(padded).reshape(_B, _T, _V)


# speedup vs baseline: 1.4092x; 1.2882x over previous
import jax
import jax.numpy as jnp
from jax.experimental import pallas as pl
from jax.experimental.pallas import tpu as pltpu
from jax.experimental.pallas import tpu_sc as plsc

_B, _T, _V = 1024, 50, 1000
_VP = 1024
_N = _B * _T
_W = 40
_R = 512

_MESH = plsc.VectorSubcoreMesh(core_axis_name="c", subcore_axis_name="s")


def _sc_gather(tabp, idx3):
    @pl.kernel(
        out_type=jax.ShapeDtypeStruct((_N, _VP), tabp.dtype),
        mesh=_MESH,
    )
    def gather_kernel(table_hbm, idx_hbm, out_hbm):
        def body(idx_vmem, out_vmem):
            pltpu.sync_copy(table_hbm.at[idx_vmem.at[0, 0]], out_vmem)

        pltpu.emit_pipeline(
            body,
            grid=(_N // _W,),
            in_specs=[pl.BlockSpec((1, 1, _W), lambda i: (i, 0, 0))],
            out_specs=[pl.BlockSpec((_W, _VP), lambda i: (i, 0))],
            core_axis_name=("c", "s"),
            dimension_semantics=(pltpu.PARALLEL,),
        )(idx_hbm, out_hbm)

    return gather_kernel(tabp, idx3)


def _tc_depad(padded):
    return pl.pallas_call(
        lambda x_ref, o_ref: o_ref.__setitem__(
            (slice(None), slice(None)), x_ref[:, :_V]
        ),
        out_shape=jax.ShapeDtypeStruct((_N, _V), padded.dtype),
        grid=(_N // _R,),
        in_specs=[pl.BlockSpec((_R, _VP), lambda i: (i, 0))],
        out_specs=pl.BlockSpec((_R, _V), lambda i: (i, 0)),
    )(padded)


def kernel(idx, table):
    tabp = jnp.pad(table, ((0, 0), (0, _VP - _V)))
    idx3 = idx.reshape(_N // _W, 1, _W)
    padded = _sc_gather(tabp, idx3)
    return padded[:, :_V].reshape(_B, _T, _V)
